# baseline (speedup 1.0000x reference)
import jax
import jax.numpy as jnp
from jax.experimental import pallas as pl

HID = 16
E = 6
MAXR = 4
SCALE = 0.95
OUTF = 10
BB = 16


def _taps9(hp, bb, n, c):
    cols = [hp[:, dy:dy + n, dx:dx + n, :].reshape(bb, n * n, c)
            for dy in range(3) for dx in range(3)]
    return jnp.concatenate(cols, axis=-1).reshape(bb * n * n, 9 * c)


def _body(X1T, W1m, b1r, Wall, ball, Wgc, Dmask, Wlm, blr,
          Wf1r, bf1r, Wf2T, bf2r, out_ref):
    x1 = X1T[...]
    dn = (((0,), (0,)), ((), ()))
    hs = [jax.lax.dot_general(x1[s], W1m[...], dn,
                              preferred_element_type=jnp.float32)
          for s in range(BB)]
    h = jax.nn.relu(jnp.stack(hs, axis=0) + b1r[...])

    Wg = Wgc[...]
    dmask = Dmask[...]
    dnc0 = (((0,), (0,)), ((), ()))
    for _ in range(MAXR):
        Ms = [jax.lax.dot_general(h[s], Wg, dnc0,
                                  preferred_element_type=jnp.float32)
              for s in range(BB)]
        Md = jnp.stack(Ms, axis=0) * dmask[None]
        cs = jnp.sum(Md, axis=1)
        logits = jnp.concatenate(
            [jnp.sum(cs[:, e * HID:(e + 1) * HID], axis=1, keepdims=True)
             for e in range(E)], axis=1)[:, None, :]
        mx = jnp.max(logits, axis=2, keepdims=True)
        ismax = logits == mx
        iota = jax.lax.broadcasted_iota(jnp.int32, (BB, 1, E), 2)
        first_idx = jnp.min(jnp.where(ismax, iota, E), axis=2, keepdims=True)
        w = (iota == first_idx).astype(jnp.float32) * SCALE

        hp = jnp.pad(h.reshape(BB, 16, 16, HID),
                     ((0, 0), (1, 1), (1, 1), (0, 0)))
        X = _taps9(hp, BB, 16, HID)
        Y = jax.nn.relu(
            jnp.dot(X, Wall[...], preferred_element_type=jnp.float32) + ball[...])
        Y = Y.reshape(BB, 256, E * HID)
        h = sum(Y[:, :, e * HID:(e + 1) * HID] * w[:, :, e:e + 1]
                for e in range(E))

    hp = jnp.pad(h.reshape(BB, 16, 16, HID), ((0, 0), (1, 1), (1, 1), (0, 0)))
    Xl = _taps9(hp, BB, 16, HID)
    c1 = jax.nn.relu(
        jnp.dot(Xl, Wlm[...], preferred_element_type=jnp.float32) + blr[...])
    c1 = c1.reshape(BB, 4, 4, 16, HID // 2)
    my = jnp.maximum(c1[:, :, 0], c1[:, :, 2])
    m = jnp.maximum(my[:, :, 0:14, :], my[:, :, 2:16, :])
    pooled = jnp.concatenate([m[:, :, 4 * X:4 * X + 1, :] for X in range(4)],
                             axis=2)
    pr = pooled.reshape(BB, 16, HID // 2)
    Wf1 = Wf1r[...]
    f_parts = [jnp.sum(pr * Wf1[o][None], axis=(1, 2), keepdims=True)
               for o in range(HID)]
    f1 = jnp.concatenate(f_parts, axis=2).reshape(BB, HID)
    f1 = jax.nn.relu(f1 + bf1r[...])
    out_ref[...] = jnp.dot(f1, Wf2T[...],
                           preferred_element_type=jnp.float32) + bf2r[...]


def kernel(x, W1, b1, Wg, We, be, Wl, bl, Wf1, bf1, Wf2, bf2):
    B = x.shape[0]
    xp = jnp.pad(x.transpose(0, 2, 3, 1), ((0, 0), (1, 1), (1, 1), (0, 0)))
    ph = ((xp[:, 0::2, 0::2, :], xp[:, 0::2, 1::2, :]),
          (xp[:, 1::2, 0::2, :], xp[:, 1::2, 1::2, :]))
    taps = []
    for dy in range(3):
        for dx in range(3):
            t = ph[dy % 2][dx % 2][:, dy // 2:dy // 2 + 16,
                                   dx // 2:dx // 2 + 16, :]
            taps.append(t.transpose(0, 3, 1, 2).reshape(B, 3, 256))
    X1T = jnp.concatenate(taps, axis=1)
    W1m = W1.transpose(2, 3, 1, 0).reshape(27, HID)
    Wall = We.transpose(3, 4, 2, 0, 1).reshape(9 * HID, E * HID)
    ball = be.reshape(1, E * HID)
    Wgr = Wg.reshape(E, HID, 16, 16).transpose(0, 2, 3, 1).reshape(E, 256, HID)
    Wgc = Wgr.transpose(1, 0, 2).reshape(256, E * HID)
    Dmask = (jnp.arange(E * HID)[None, :] % HID ==
             jnp.arange(HID)[:, None]).astype(jnp.float32)
    Wlm = Wl.transpose(2, 3, 1, 0).reshape(9 * HID, HID // 2)
    Wf1r = Wf1.reshape(HID, HID // 2, 4, 4).transpose(0, 2, 3, 1).reshape(
        HID, 16, HID // 2)
    Wf2T = Wf2.T

    grid = (B // BB,)
    bspec = lambda shp: pl.BlockSpec(shp, lambda i: (0,) * len(shp))
    out = pl.pallas_call(
        _body,
        grid=grid,
        in_specs=[
            pl.BlockSpec((BB, 27, 256), lambda i: (i, 0, 0)),
            bspec((27, HID)),
            bspec((1, 1, HID)),
            bspec((9 * HID, E * HID)),
            bspec((1, E * HID)),
            bspec((256, E * HID)),
            bspec((HID, E * HID)),
            bspec((9 * HID, HID // 2)),
            bspec((1, HID // 2)),
            bspec((HID, 16, HID // 2)),
            bspec((1, HID)),
            bspec((HID, OUTF)),
            bspec((1, OUTF)),
        ],
        out_specs=pl.BlockSpec((BB, OUTF), lambda i: (i, 0)),
        out_shape=jax.ShapeDtypeStruct((B, OUTF), jnp.float32),
    )(X1T, W1m, b1.reshape(1, 1, HID), Wall, ball, Wgc, Dmask, Wlm,
      bl.reshape(1, HID // 2), Wf1r, bf1.reshape(1, HID), Wf2T,
      bf2.reshape(1, OUTF))
    return out
